# baseline (speedup 1.0000x reference)
import jax
import jax.numpy as jnp
from jax import lax
from jax.experimental import pallas as pl
from jax.experimental.pallas import tpu as pltpu
from jax.experimental.pallas import tpu_sc as plsc


def kernel(token_ids, embeddings):
    batch, seq = token_ids.shape
    dim = embeddings.shape[1]
    nj = batch // 128
    ni = dim // 8
    ids_t = token_ids.T

    mesh = plsc.VectorSubcoreMesh(core_axis_name="core",
                                  subcore_axis_name="subcore")

    @pl.kernel(
        out_type=jax.ShapeDtypeStruct((seq, ni, nj, 8, 128), embeddings.dtype),
        mesh=mesh,
        compiler_params=pltpu.CompilerParams(use_tc_tiling_on_sc=False,
                                             needs_layout_passes=False),
    )
    def sc_gather(table_hbm, ids_hbm, out_hbm):
        def body(ids_vmem, out_vmem):
            def inner(buf):
                pltpu.sync_copy(table_hbm.at[ids_vmem.at[0]], buf)
                lane = lax.iota(jnp.int32, 16)
                for i in range(ni):
                    for c in range(8):
                        d = jnp.full((16,), i * 8 + c, jnp.int32)

                        @pl.loop(0, 8)
                        def _(k):
                            rows = k * 16 + lane
                            vals = plsc.load_gather(buf, [rows, d])
                            out_vmem[0, i, 0, c, pl.ds(k * 16, 16)] = vals

            pl.run_scoped(inner, pltpu.VMEM((128, dim), embeddings.dtype))

        pltpu.emit_pipeline(
            body,
            grid=(seq, nj),
            in_specs=[pl.BlockSpec((1, 128), index_map=lambda s, j: (s, j))],
            out_specs=[pl.BlockSpec((1, ni, 1, 8, 128),
                                    index_map=lambda s, j: (s, 0, j, 0, 0))],
            core_axis_name=("core", "subcore"),
            dimension_semantics=(pltpu.PARALLEL, pltpu.PARALLEL),
        )(ids_hbm, out_hbm)

    o5 = sc_gather(embeddings, ids_t)
    return o5.transpose(2, 4, 0, 1, 3).reshape(batch, seq, dim)


# speedup vs baseline: 1.3578x; 1.3578x over previous
import jax
import jax.numpy as jnp
from jax import lax
from jax.experimental import pallas as pl
from jax.experimental.pallas import tpu as pltpu
from jax.experimental.pallas import tpu_sc as plsc

_TOK = 256
_CH = 32
_NCH = _TOK // _CH


def kernel(token_ids, embeddings):
    batch, seq = token_ids.shape
    dim = embeddings.shape[1]
    nj = batch // 128
    ni = dim // 8
    jb = _TOK // 128
    ids_t = token_ids.T

    mesh = plsc.VectorSubcoreMesh(core_axis_name="core",
                                  subcore_axis_name="subcore")

    @pl.kernel(
        out_type=jax.ShapeDtypeStruct((seq, ni, nj, 8, 128), embeddings.dtype),
        mesh=mesh,
        compiler_params=pltpu.CompilerParams(use_tc_tiling_on_sc=False,
                                             needs_layout_passes=False),
    )
    def sc_gather(table_hbm, ids_hbm, out_hbm):
        def body(ids_vmem, out_vmem):
            def inner(*scoped):
                bufs, sems = scoped[:_NCH], scoped[_NCH:]
                copies = []
                for q in range(_NCH):
                    copies.append(pltpu.async_copy(
                        table_hbm.at[ids_vmem.at[0, pl.ds(q * _CH, _CH)]],
                        bufs[q], sems[q]))
                lane = lax.iota(jnp.int32, 16)
                for q in range(_NCH):
                    copies[q].wait()
                    buf = bufs[q]
                    jq = q // 4
                    lq = q % 4

                    @plsc.parallel_loop(0, 2 * ni * 8, unroll=8)
                    def _(idx):
                        i = idx >> 4
                        c = (idx >> 1) & 7
                        kl = idx & 1
                        rows = kl * 16 + lane
                        d = jnp.full((16,), i * 8 + c, jnp.int32)
                        vals = plsc.load_gather(buf, [rows, d])
                        out_vmem[0, i, jq, c,
                                 pl.ds((lq * 2 + kl) * 16, 16)] = vals

            pl.run_scoped(inner,
                          *[pltpu.VMEM((_CH, dim), embeddings.dtype)
                            for _ in range(_NCH)],
                          *[pltpu.SemaphoreType.DMA for _ in range(_NCH)])

        pltpu.emit_pipeline(
            body,
            grid=(seq, nj // jb),
            in_specs=[pl.BlockSpec((1, _TOK), index_map=lambda s, j: (s, j))],
            out_specs=[pl.BlockSpec((1, ni, jb, 8, 128),
                                    index_map=lambda s, j: (s, 0, j, 0, 0))],
            core_axis_name=("core", "subcore"),
            dimension_semantics=(pltpu.PARALLEL, pltpu.PARALLEL),
        )(ids_hbm, out_hbm)

    o5 = sc_gather(embeddings, ids_t)
    return o5.transpose(2, 4, 0, 1, 3).reshape(batch, seq, dim)


# speedup vs baseline: 1.4027x; 1.0331x over previous
import jax
import jax.numpy as jnp
from jax import lax
from jax.experimental import pallas as pl
from jax.experimental.pallas import tpu as pltpu
from jax.experimental.pallas import tpu_sc as plsc

_TOK = 512
_CH = 32
_NCH = _TOK // _CH


def kernel(token_ids, embeddings):
    batch, seq = token_ids.shape
    dim = embeddings.shape[1]
    nj = batch // 128
    ni = dim // 8
    jb = _TOK // 128
    ids_t = token_ids.T

    mesh = plsc.VectorSubcoreMesh(core_axis_name="core",
                                  subcore_axis_name="subcore")

    @pl.kernel(
        out_type=jax.ShapeDtypeStruct((seq, ni, nj, 8, 128), embeddings.dtype),
        mesh=mesh,
        compiler_params=pltpu.CompilerParams(use_tc_tiling_on_sc=False,
                                             needs_layout_passes=False),
    )
    def sc_gather(table_hbm, ids_hbm, out_hbm):
        def body(ids_vmem, out_vmem):
            def inner(*scoped):
                bufs, sems = scoped[:_NCH], scoped[_NCH:]
                copies = []
                for q in range(_NCH):
                    copies.append(pltpu.async_copy(
                        table_hbm.at[ids_vmem.at[0, pl.ds(q * _CH, _CH)]],
                        bufs[q], sems[q]))
                lane = lax.iota(jnp.int32, 16)
                for q in range(_NCH):
                    copies[q].wait()
                    buf = bufs[q]
                    jq = q // 4
                    lq = q % 4

                    @plsc.parallel_loop(0, 2 * ni * 8, unroll=8)
                    def _(idx):
                        i = idx >> 4
                        c = (idx >> 1) & 7
                        kl = idx & 1
                        rows = kl * 16 + lane
                        d = jnp.full((16,), i * 8 + c, jnp.int32)
                        vals = plsc.load_gather(buf, [rows, d])
                        out_vmem[0, i, jq, c,
                                 pl.ds((lq * 2 + kl) * 16, 16)] = vals

            pl.run_scoped(inner,
                          *[pltpu.VMEM((_CH, dim), embeddings.dtype)
                            for _ in range(_NCH)],
                          *[pltpu.SemaphoreType.DMA for _ in range(_NCH)])

        pltpu.emit_pipeline(
            body,
            grid=(seq, nj // jb),
            in_specs=[pl.BlockSpec((1, _TOK), index_map=lambda s, j: (s, j))],
            out_specs=[pl.BlockSpec((1, ni, jb, 8, 128),
                                    index_map=lambda s, j: (s, 0, j, 0, 0))],
            core_axis_name=("core", "subcore"),
            dimension_semantics=(pltpu.PARALLEL, pltpu.PARALLEL),
        )(ids_hbm, out_hbm)

    o5 = sc_gather(embeddings, ids_t)
    return o5.transpose(2, 4, 0, 1, 3).reshape(batch, seq, dim)
